# baseline (speedup 1.0000x reference)
import math

import jax
import jax.numpy as jnp
import numpy as np
from jax.experimental import pallas as pl
from jax.experimental.pallas import tpu as pltpu


def _pos_encoding_np(N, D):
    position = np.arange(N)[:, None].astype(np.float64)
    div_term = np.exp(np.arange(0, D, 2).astype(np.float64) * (-math.log(10000.0) / D))
    pe = np.zeros((N, D), dtype=np.float32)
    pe[:, 0::2] = np.sin(position * div_term)
    pe[:, 1::2] = np.cos(position * div_term)
    return pe


_gumbel_cache = {}


def _threefry2x32_np(k0, k1, x0, x1):
    def rol(x, d):
        return ((x << np.uint32(d)) | (x >> np.uint32(32 - d))).astype(np.uint32)
    rotations = [(13, 15, 26, 6), (17, 29, 16, 24)]
    ks = [np.uint32(k0), np.uint32(k1),
          np.uint32(np.uint32(k0) ^ np.uint32(k1) ^ np.uint32(0x1BD11BDA))]
    x0 = (x0 + ks[0]).astype(np.uint32)
    x1 = (x1 + ks[1]).astype(np.uint32)
    for i in range(5):
        for r in rotations[i % 2]:
            x0 = (x0 + x1).astype(np.uint32)
            x1 = rol(x1, r)
            x1 = (x1 ^ x0).astype(np.uint32)
        x0 = (x0 + ks[(i + 1) % 3]).astype(np.uint32)
        x1 = (x1 + ks[(i + 2) % 3] + np.uint32(i + 1)).astype(np.uint32)
    return x0, x1


def _gumbel_const(B, N):
    if (B, N) not in _gumbel_cache:
        n = B * N * 2
        c = np.arange(n, dtype=np.uint32)
        if jax.config.jax_threefry_partitionable:
            x0, x1 = _threefry2x32_np(0, 1, np.zeros(n, np.uint32), c)
            bits = (x0 ^ x1).astype(np.uint32)
        else:
            x0, x1 = _threefry2x32_np(0, 1, c[:n // 2], c[n // 2:])
            bits = np.concatenate([x0, x1])
        f = ((bits >> np.uint32(9)) | np.uint32(0x3F800000)).view(np.float32) \
            - np.float32(1.0)
        u = np.maximum(np.float32(1e-20),
                       f * (np.float32(1.0) - np.float32(1e-20))
                       + np.float32(1e-20))
        g = (-np.log(-np.log(u.astype(np.float64)))).astype(np.float32)
        _gumbel_cache[(B, N)] = g.reshape(B, N, 2)
    return _gumbel_cache[(B, N)]


def _adj_body(ids_ref, latent_ref, action_ref, wmask_ref, bmask_ref, pos_ref,
              g_ref, w10_ref, w1e_ref, b1_ref, w2_ref, b2_ref, ones_ref,
              out_ref, t_scr):
    N, D = latent_ref.shape[-2], latent_ref.shape[-1]
    A = action_ref.shape[-1]
    b = pl.program_id(0)
    eid = ids_ref[b] + 1
    lat = latent_ref[0]
    act = action_ref[pl.ds(b, 1), :]

    act_rep = jnp.broadcast_to(act, (N, A))
    ap = jnp.concatenate([act_rep, pos_ref[...]], axis=-1)
    inter_mask = jax.nn.sigmoid(
        jnp.dot(ap, wmask_ref[...], preferred_element_type=jnp.float32)
        + bmask_ref[...])
    s = jnp.sum(lat * inter_mask, axis=-1, keepdims=True)
    l0 = jnp.log(jnp.maximum(1.0 - s, 0.0001))
    l1 = jnp.log(jnp.maximum(s, 0.0001))
    g = g_ref[0]
    z0 = l0 + g[:, 0:1]
    z1 = l1 + g[:, 1:2]
    zm = jnp.maximum(z0, z1)
    e0 = jnp.exp(z0 - zm)
    e1 = jnp.exp(z1 - zm)
    esum = e0 + e1
    y0 = e0 / esum
    y1 = e1 / esum
    sel = y1 > y0

    w10 = w10_ref[0]
    w1e = w1e_ref[0]
    u0 = jnp.dot(lat, w10[:D], preferred_element_type=jnp.float32)
    v0 = jnp.dot(lat, w10[D:], preferred_element_type=jnp.float32) \
        + b1_ref[0:1, :]
    ue = jnp.dot(lat, w1e[:D], preferred_element_type=jnp.float32)
    ve = jnp.dot(lat, w1e[D:], preferred_element_type=jnp.float32) \
        + b1_ref[pl.ds(eid, 1), :]
    H = u0.shape[-1]
    Hh = H // 2

    usel = jnp.where(sel, ue, u0)
    w2sel = jnp.where(sel, w2_ref[pl.ds(eid, 1), :],
                      w2_ref[0:1, :])
    b2sel = jnp.where(sel, b2_ref[pl.ds(eid, 1), 0:1],
                      b2_ref[0:1, 0:1])
    sel3 = sel[:, None, :]
    vsel = jnp.where(jnp.broadcast_to(sel3, (N, N, H)),
                     jnp.broadcast_to(ve[None, :, :], (N, N, H)),
                     jnp.broadcast_to(v0[None, :, :], (N, N, H)))
    big = usel[:, None, :] + vsel
    h = jnp.maximum(big, 0.01 * big)
    hws = (h[..., :Hh] * w2sel[:, None, :Hh]
           + h[..., Hh:] * w2sel[:, None, Hh:])
    t = jnp.dot(hws.reshape(N * N, Hh), ones_ref[...],
                preferred_element_type=jnp.float32)
    t_scr[...] = t.reshape(N, N)
    out_ref[0] = jax.nn.sigmoid(t_scr[...] + b2sel)


def kernel(latent, action, W_mask, b_mask, W1, b1, W2, b2):
    B, N, D = latent.shape
    A = action.shape[-1]
    H = W1.shape[-1]
    ids = jnp.argmax(action, axis=-1).astype(jnp.int32)
    pos = _pos_encoding_np(N, D)
    g = _gumbel_const(B, N)
    ones = np.ones((H // 2, 1), dtype=np.float32)

    grid_spec = pltpu.PrefetchScalarGridSpec(
        num_scalar_prefetch=1,
        grid=(B,),
        in_specs=[
            pl.BlockSpec((1, N, D), lambda b, ids: (b, 0, 0)),
            pl.BlockSpec((B, A), lambda b, ids: (0, 0)),
            pl.BlockSpec((A + D, D), lambda b, ids: (0, 0)),
            pl.BlockSpec((D,), lambda b, ids: (0,)),
            pl.BlockSpec((N, D), lambda b, ids: (0, 0)),
            pl.BlockSpec((1, N, 2), lambda b, ids: (b, 0, 0)),
            pl.BlockSpec((1, 2 * D, H), lambda b, ids: (0, 0, 0)),
            pl.BlockSpec((1, 2 * D, H), lambda b, ids: (ids[b] + 1, 0, 0)),
            pl.BlockSpec((A + 1, H), lambda b, ids: (0, 0)),
            pl.BlockSpec((A + 1, H), lambda b, ids: (0, 0)),
            pl.BlockSpec((A + 1, 1), lambda b, ids: (0, 0)),
            pl.BlockSpec((H // 2, 1), lambda b, ids: (0, 0)),
        ],
        out_specs=pl.BlockSpec((1, N, N), lambda b, ids: (b, 0, 0)),
        scratch_shapes=[pltpu.VMEM((N, N), jnp.float32)],
    )
    return pl.pallas_call(
        _adj_body,
        grid_spec=grid_spec,
        out_shape=jax.ShapeDtypeStruct((B, N, N), jnp.float32),
        compiler_params=pltpu.CompilerParams(dimension_semantics=("arbitrary",)),
    )(ids, latent, action, W_mask, b_mask, pos, g, W1, W1, b1,
      jnp.squeeze(W2, -1), b2, ones)


# speedup vs baseline: 1.1555x; 1.1555x over previous
import math

import jax
import jax.numpy as jnp
import numpy as np
from jax.experimental import pallas as pl
from jax.experimental.pallas import tpu as pltpu


def _pos_encoding_np(N, D):
    position = np.arange(N)[:, None].astype(np.float64)
    div_term = np.exp(np.arange(0, D, 2).astype(np.float64) * (-math.log(10000.0) / D))
    pe = np.zeros((N, D), dtype=np.float32)
    pe[:, 0::2] = np.sin(position * div_term)
    pe[:, 1::2] = np.cos(position * div_term)
    return pe


_gumbel_cache = {}


def _threefry2x32_np(k0, k1, x0, x1):
    def rol(x, d):
        return ((x << np.uint32(d)) | (x >> np.uint32(32 - d))).astype(np.uint32)
    rotations = [(13, 15, 26, 6), (17, 29, 16, 24)]
    ks = [np.uint32(k0), np.uint32(k1),
          np.uint32(np.uint32(k0) ^ np.uint32(k1) ^ np.uint32(0x1BD11BDA))]
    x0 = (x0 + ks[0]).astype(np.uint32)
    x1 = (x1 + ks[1]).astype(np.uint32)
    for i in range(5):
        for r in rotations[i % 2]:
            x0 = (x0 + x1).astype(np.uint32)
            x1 = rol(x1, r)
            x1 = (x1 ^ x0).astype(np.uint32)
        x0 = (x0 + ks[(i + 1) % 3]).astype(np.uint32)
        x1 = (x1 + ks[(i + 2) % 3] + np.uint32(i + 1)).astype(np.uint32)
    return x0, x1


def _gumbel_const(B, N):
    if (B, N) not in _gumbel_cache:
        n = B * N * 2
        c = np.arange(n, dtype=np.uint32)
        if jax.config.jax_threefry_partitionable:
            x0, x1 = _threefry2x32_np(0, 1, np.zeros(n, np.uint32), c)
            bits = (x0 ^ x1).astype(np.uint32)
        else:
            x0, x1 = _threefry2x32_np(0, 1, c[:n // 2], c[n // 2:])
            bits = np.concatenate([x0, x1])
        f = ((bits >> np.uint32(9)) | np.uint32(0x3F800000)).view(np.float32) \
            - np.float32(1.0)
        u = np.maximum(np.float32(1e-20),
                       f * (np.float32(1.0) - np.float32(1e-20))
                       + np.float32(1e-20))
        g = (-np.log(-np.log(u.astype(np.float64)))).astype(np.float32)
        _gumbel_cache[(B, N)] = g.reshape(B, N, 2)
    return _gumbel_cache[(B, N)]


def _adj_body(ids_ref, latent_ref, action_ref, wmask_ref, bmask_ref, pos_ref,
              g_ref, w10_ref, w1e_ref, b1_ref, w20_ref, w2e_ref,
              b2_ref, out_ref, t0_scr, te_scr):
    N, D = latent_ref.shape[-2], latent_ref.shape[-1]
    A = action_ref.shape[-1]
    b = pl.program_id(0)
    eid = ids_ref[b] + 1
    lat = latent_ref[0]
    act = action_ref[pl.ds(b, 1), :]

    act_rep = jnp.broadcast_to(act, (N, A))
    ap = jnp.concatenate([act_rep, pos_ref[...]], axis=-1)
    inter_mask = jax.nn.sigmoid(
        jnp.dot(ap, wmask_ref[...], preferred_element_type=jnp.float32)
        + bmask_ref[...])
    s = jnp.sum(lat * inter_mask, axis=-1, keepdims=True)
    l0 = jnp.log(jnp.maximum(1.0 - s, 0.0001))
    l1 = jnp.log(jnp.maximum(s, 0.0001))
    g = g_ref[0]
    z0 = l0 + g[:, 0:1]
    z1 = l1 + g[:, 1:2]
    zm = jnp.maximum(z0, z1)
    e0 = jnp.exp(z0 - zm)
    e1 = jnp.exp(z1 - zm)
    esum = e0 + e1
    y0 = e0 / esum
    y1 = e1 / esum
    hard = (y1 > y0).astype(jnp.float32)
    mask = hard + y1 - y1

    def expert(w1_ref, b1row, w2_ref, scr):
        w1 = w1_ref[0]
        H = w1.shape[-1]
        u = jnp.dot(lat, w1[:D], preferred_element_type=jnp.float32)
        v = jnp.dot(lat, w1[D:], preferred_element_type=jnp.float32)
        big = u[:, None, :] + v[None, :, :] + b1row[None, :, :]
        h = jnp.maximum(big, 0.01 * big)
        t = jnp.dot(h.reshape(N * N, H), w2_ref[0],
                    preferred_element_type=jnp.float32)
        scr[...] = t.reshape(N, N)
        return scr[...]

    t0 = expert(w10_ref, b1_ref[0:1, :], w20_ref, t0_scr) + b2_ref[0:1, 0:1]
    te = expert(w1e_ref, b1_ref[pl.ds(eid, 1), :], w2e_ref, te_scr) \
        + b2_ref[pl.ds(eid, 1), 0:1]
    c0 = jax.nn.sigmoid(t0)
    ce = jax.nn.sigmoid(te)
    out_ref[0] = c0 * (1.0 - mask) + ce * mask


def kernel(latent, action, W_mask, b_mask, W1, b1, W2, b2):
    B, N, D = latent.shape
    A = action.shape[-1]
    H = W1.shape[-1]
    ids = jnp.argmax(action, axis=-1).astype(jnp.int32)
    pos = _pos_encoding_np(N, D)
    g = _gumbel_const(B, N)

    grid_spec = pltpu.PrefetchScalarGridSpec(
        num_scalar_prefetch=1,
        grid=(B,),
        in_specs=[
            pl.BlockSpec((1, N, D), lambda b, ids: (b, 0, 0)),
            pl.BlockSpec((B, A), lambda b, ids: (0, 0)),
            pl.BlockSpec((A + D, D), lambda b, ids: (0, 0)),
            pl.BlockSpec((1, D), lambda b, ids: (0, 0)),
            pl.BlockSpec((N, D), lambda b, ids: (0, 0)),
            pl.BlockSpec((1, N, 2), lambda b, ids: (b, 0, 0)),
            pl.BlockSpec((1, 2 * D, H), lambda b, ids: (0, 0, 0)),
            pl.BlockSpec((1, 2 * D, H), lambda b, ids: (ids[b] + 1, 0, 0)),
            pl.BlockSpec((A + 1, H), lambda b, ids: (0, 0)),
            pl.BlockSpec((1, H, 1), lambda b, ids: (0, 0, 0)),
            pl.BlockSpec((1, H, 1), lambda b, ids: (ids[b] + 1, 0, 0)),
            pl.BlockSpec((A + 1, 1), lambda b, ids: (0, 0)),
        ],
        out_specs=pl.BlockSpec((1, N, N), lambda b, ids: (b, 0, 0)),
        scratch_shapes=[pltpu.VMEM((N, N), jnp.float32),
                        pltpu.VMEM((N, N), jnp.float32)],
    )
    return pl.pallas_call(
        _adj_body,
        grid_spec=grid_spec,
        out_shape=jax.ShapeDtypeStruct((B, N, N), jnp.float32),
        compiler_params=pltpu.CompilerParams(dimension_semantics=("arbitrary",)),
    )(ids, latent, action, W_mask, b_mask.reshape(1, D), pos,
      g, W1, W1, b1, W2, W2, b2)
